# baseline (speedup 1.0000x reference)
import functools

import jax
import jax.numpy as jnp
from jax import lax
from jax.experimental import pallas as pl
from jax.experimental.pallas import tpu as pltpu
from jax.experimental.pallas import tpu_sc as plsc

N_NODES = 10000
N_EDGES = 320000
IN_DIM = 128
OUT_DIM = 128
NUM_REL = 64
NRH = NUM_REL // 2

H = 64
NC = 2
NS = 16
LANES = 16

EPW = N_EDGES // NS
B = 80
NCHUNK = EPW // B
NSC = 10
NSUP = NCHUNK // NSC
ZROWS_A = 1256
ZROWS_B = 2 * N_NODES - (NS - 1) * ZROWS_A
ZB = 8


def _sc_accumulate(xh, relf, src3, dst3, typ3, nrm3, zrs):
  mesh = plsc.VectorSubcoreMesh(core_axis_name="c", subcore_axis_name="s")

  @functools.partial(
      pl.kernel,
      out_type=jax.ShapeDtypeStruct((NC, 2 * N_NODES, H), jnp.float32),
      mesh=mesh,
      compiler_params=pltpu.CompilerParams(use_tc_tiling_on_sc=False),
      scratch_types=[
          pltpu.VMEM((2, NSC, B), jnp.int32),
          pltpu.VMEM((2, NSC, B), jnp.int32),
          pltpu.VMEM((2, NSC, B), jnp.int32),
          pltpu.VMEM((2, NSC, B), jnp.float32),
          pltpu.VMEM((2, B, H), jnp.float32),
          pltpu.VMEM((2, B, H), jnp.float32),
          pltpu.VMEM((2, B), jnp.int32),
          pltpu.VMEM((2, B, H), jnp.float32),
          pltpu.VMEM_SHARED((2 * NRH, H), jnp.float32),
          pltpu.VMEM_SHARED((2 * N_NODES, H), jnp.float32),
          pltpu.SemaphoreType.DMA,
          pltpu.SemaphoreType.DMA,
          pltpu.SemaphoreType.DMA,
          pltpu.SemaphoreType.DMA,
          pltpu.SemaphoreType.DMA,
          pltpu.SemaphoreType.DMA,
          pltpu.SemaphoreType.DMA,
          pltpu.SemaphoreType.DMA,
      ],
  )
  def sc_kernel(xh_hbm, rel_hbm, src_hbm, dst_hbm, typ_hbm, nrm_hbm, zrs_hbm,
                out_hbm, gidx_v, sidx_v, typ_v, nrm_v, xrows_v, comp_v, ridx_v,
                rrows_v, rel_sh, acc_sh, msem0, msem1, gsem0, gsem1, rsem0,
                rsem1, ssem0, ssem1):
    c = lax.axis_index("c")
    s = lax.axis_index("s")
    msems = (msem0, msem1)
    gsems = (gsem0, gsem1)
    rsems = (rsem0, rsem1)
    ssems = (ssem0, ssem1)

    def issue_meta(sb, m):
      pltpu.async_copy(src_hbm.at[s, sb], gidx_v.at[m], msems[m])
      pltpu.async_copy(dst_hbm.at[s, sb], sidx_v.at[m], msems[m])
      pltpu.async_copy(typ_hbm.at[s, sb], typ_v.at[m], msems[m])
      pltpu.async_copy(nrm_hbm.at[s, sb], nrm_v.at[m], msems[m])

    def wait_meta(sb, m):
      for _ in range(4):
        pltpu.make_async_copy(src_hbm.at[s, sb], gidx_v.at[m], msems[m]).wait()

    issue_meta(0, 0)

    zbase = s * ZROWS_A

    @pl.when(s < NS - 1)
    def _():
      pltpu.sync_copy(zrs_hbm, acc_sh.at[pl.ds(zbase, ZROWS_A)])

    @pl.when(s == NS - 1)
    def _():
      pltpu.sync_copy(zrs_hbm.at[pl.ds(0, ZROWS_B)],
                      acc_sh.at[pl.ds(zbase, ZROWS_B)])

    @pl.when(s == 0)
    def _():
      pltpu.sync_copy(rel_hbm, rel_sh)


    def fix_gidx(m, g):
      for k in range(B // LANES):
        sl = pl.ds(k * LANES, LANES)
        gidx_v[m, g, sl] = gidx_v[m, g, sl] * 2 + c

    def fix_sidx(m, g):
      for k in range(B // LANES):
        sl = pl.ds(k * LANES, LANES)
        sidx_v[m, g, sl] = sidx_v[m, g, sl] + ((typ_v[m, g, sl] >> 5) * N_NODES)

    def issue_gather(m, g, b):
      pltpu.async_copy(xh_hbm.at[gidx_v.at[m, g]], xrows_v.at[b], gsems[b])

    def issue_rel_gather(m, g, b):
      for k in range(B // LANES):
        sl = pl.ds(k * LANES, LANES)
        ridx_v[b, sl] = (typ_v[m, g, sl] & (NRH - 1)) * 2 + c
      pltpu.async_copy(rel_sh.at[ridx_v.at[b]], rrows_v.at[b], rsems[b])

    plsc.subcore_barrier()

    def prime_super(m):
      for b in range(2):
        fix_gidx(m, b)
        issue_gather(m, b, b)
        issue_rel_gather(m, b, b)

    def drain_scatter(m, b):
      pltpu.make_async_copy(comp_v.at[b], acc_sh.at[sidx_v.at[m, 0]],
                            ssems[b]).wait()

    def super_body(sb, m, first):
      @pl.loop(0, NSC, step=2)
      def _(g):
        @pl.when(jnp.logical_and(g == 2, sb + 1 < NSUP))
        def _():
          issue_meta(sb + 1, 1 - m)

        for b in range(2):
          gb = g + b
          pltpu.make_async_copy(xh_hbm.at[gidx_v.at[m, gb]], xrows_v.at[b],
                                gsems[b]).wait()
          if first:
            @pl.when(gb >= 2)
            def _():
              drain_scatter(m, b)
          else:
            drain_scatter(m, b)

          pltpu.make_async_copy(rel_sh.at[ridx_v.at[b]], rrows_v.at[b],
                                rsems[b]).wait()

          @pl.loop(0, B // LANES, unroll=B // LANES)
          def _(k):
            nvec = nrm_v[m, gb, pl.ds(k * LANES, LANES)]
            for l in range(LANES):
              nv = lax.gather(
                  nvec, jnp.full((LANES, 1), l, jnp.int32),
                  lax.GatherDimensionNumbers(offset_dims=(),
                                             collapsed_slice_dims=(0,),
                                             start_index_map=(0,)),
                  slice_sizes=(1,),
                  mode=lax.GatherScatterMode.PROMISE_IN_BOUNDS)
              i = k * LANES + l
              xs = [xrows_v[b, i, pl.ds(j * LANES, LANES)]
                    for j in range(H // LANES)]
              rs = [rrows_v[b, i, pl.ds(j * LANES, LANES)]
                    for j in range(H // LANES)]
              for j in range(H // LANES):
                comp_v[b, i, pl.ds(j * LANES, LANES)] = (xs[j] * nv) * rs[j]

          @pl.when(gb + 2 < NSC)
          def _():
            fix_gidx(m, gb + 2)
            issue_gather(m, gb + 2, b)
            issue_rel_gather(m, gb + 2, b)

          fix_sidx(m, gb)
          pltpu.async_copy(comp_v.at[b], acc_sh.at[sidx_v.at[m, gb]], ssems[b],
                           add=True)

      @pl.when(sb + 1 < NSUP)
      def _():
        wait_meta(sb + 1, 1 - m)
        prime_super(1 - m)

    wait_meta(0, 0)
    prime_super(0)
    super_body(0, 0, True)

    @pl.loop(1, NSUP - 1, step=2)
    def _(S):
      super_body(S, 1, False)
      super_body(S + 1, 0, False)

    for b in range(2):
      drain_scatter(0, b)

    plsc.subcore_barrier()

    @pl.when(s < NS - 1)
    def _():
      pltpu.sync_copy(acc_sh.at[pl.ds(zbase, ZROWS_A)],
                      out_hbm.at[c, pl.ds(zbase, ZROWS_A)])

    @pl.when(s == NS - 1)
    def _():
      pltpu.sync_copy(acc_sh.at[pl.ds((NS - 1) * ZROWS_A, ZROWS_B)],
                      out_hbm.at[c, pl.ds((NS - 1) * ZROWS_A, ZROWS_B)])

  return sc_kernel(xh, relf, src3, dst3, typ3, nrm3, zrs)


BM = 5000


def _tc_matmul_body(acc_ref, x_ref, w4_ref, wl_ref, o_ref):
  r = jnp.dot(x_ref[...], wl_ref[...], preferred_element_type=jnp.float32)
  for k in range(4):
    r = r + jnp.dot(acc_ref[k], w4_ref[k], preferred_element_type=jnp.float32)

  o_ref[...] = r


def _tc_matmul(acc4, x, w4, w_loop):
  grid = (N_NODES // BM,)
  return pl.pallas_call(
      _tc_matmul_body,
      grid=grid,
      in_specs=[
          pl.BlockSpec((4, BM, H), lambda m: (0, m, 0)),
          pl.BlockSpec((BM, IN_DIM), lambda m: (m, 0)),
          pl.BlockSpec((4, H, OUT_DIM), lambda m: (0, 0, 0)),
          pl.BlockSpec((IN_DIM, OUT_DIM), lambda m: (0, 0)),
      ],
      out_specs=pl.BlockSpec((BM, OUT_DIM), lambda m: (m, 0)),
      out_shape=jax.ShapeDtypeStruct((N_NODES, OUT_DIM), jnp.float32),
  )(acc4, x, w4, w_loop)


@jax.jit
def kernel(x, edge_index, edge_type, edge_norm, w_loop, w_forward, w_backward,
           rel_emb):
  xh = x.reshape(2 * N_NODES, H)
  relf = rel_emb[:NRH].reshape(2 * NRH, H)
  src3 = edge_index[0].reshape(NS, NSUP, NSC, B)
  dst3 = edge_index[1].reshape(NS, NSUP, NSC, B)
  typ3 = edge_type.reshape(NS, NSUP, NSC, B)
  nrm3 = edge_norm.reshape(NS, NSUP, NSC, B)
  zrs = jnp.zeros((ZROWS_A, H), jnp.float32)

  acc = _sc_accumulate(xh, relf, src3, dst3, typ3, nrm3, zrs)
  acc4 = acc.reshape(2, 2, N_NODES, H).reshape(4, N_NODES, H)

  w4 = jnp.stack([w_forward[:H], w_backward[:H],
                  w_forward[H:], w_backward[H:]])

  return _tc_matmul(acc4, x, w4, w_loop)


# speedup vs baseline: 1.6917x; 1.6917x over previous
import functools

import jax
import jax.numpy as jnp
from jax import lax
from jax.experimental import pallas as pl
from jax.experimental.pallas import tpu as pltpu
from jax.experimental.pallas import tpu_sc as plsc

N_NODES = 10000
N_EDGES = 320000
IN_DIM = 128
OUT_DIM = 128
NUM_REL = 64
NRH = NUM_REL // 2

H = 64
NC = 2
NS = 16
LANES = 16

EPW = N_EDGES // NS
B = 80
NCHUNK = EPW // B
NSC = 10
NSUP = NCHUNK // NSC
ZROWS_A = 1256
ZROWS_B = 2 * N_NODES - (NS - 1) * ZROWS_A


def _sc_accumulate(xh, relf, src3, dst3, typ3, nrm3, zrs):
  mesh = plsc.VectorSubcoreMesh(core_axis_name="c", subcore_axis_name="s")

  @functools.partial(
      pl.kernel,
      out_type=jax.ShapeDtypeStruct((NC, 2 * N_NODES, H), jnp.float32),
      mesh=mesh,
      compiler_params=pltpu.CompilerParams(use_tc_tiling_on_sc=False),
      scratch_types=[
          pltpu.VMEM((2, NSC, B), jnp.int32),
          pltpu.VMEM((2, NSC, B), jnp.int32),
          pltpu.VMEM((2, NSC, B), jnp.int32),
          pltpu.VMEM((2, NSC, B), jnp.float32),
          pltpu.VMEM((2, B, H), jnp.float32),
          pltpu.VMEM((2, B, H), jnp.float32),
          pltpu.VMEM((2, B), jnp.int32),
          pltpu.VMEM((2, B, H), jnp.float32),
          pltpu.VMEM_SHARED((2 * NRH, H), jnp.float32),
          pltpu.VMEM_SHARED((2 * N_NODES, H), jnp.float32),
          pltpu.SemaphoreType.DMA,
          pltpu.SemaphoreType.DMA,
          pltpu.SemaphoreType.DMA,
          pltpu.SemaphoreType.DMA,
          pltpu.SemaphoreType.DMA,
          pltpu.SemaphoreType.DMA,
          pltpu.SemaphoreType.DMA,
          pltpu.SemaphoreType.DMA,
      ],
  )
  def sc_kernel(xh_hbm, rel_hbm, src_hbm, dst_hbm, typ_hbm, nrm_hbm, zrs_hbm,
                out_hbm, gidx_v, sidx_v, typ_v, nrm_v, xrows_v, comp_v, ridx_v,
                rrows_v, rel_sh, acc_sh, msem0, msem1, gsem0, gsem1, rsem0,
                rsem1, ssem0, ssem1):
    c = lax.axis_index("c")
    s = lax.axis_index("s")
    msems = (msem0, msem1)
    gsems = (gsem0, gsem1)
    rsems = (rsem0, rsem1)
    ssems = (ssem0, ssem1)

    def issue_meta(sb, m):
      pltpu.async_copy(src_hbm.at[s, sb], gidx_v.at[m], msems[m])
      pltpu.async_copy(dst_hbm.at[s, sb], sidx_v.at[m], msems[m])
      pltpu.async_copy(typ_hbm.at[s, sb], typ_v.at[m], msems[m])
      pltpu.async_copy(nrm_hbm.at[s, sb], nrm_v.at[m], msems[m])

    def wait_meta(sb, m):
      for _ in range(4):
        pltpu.make_async_copy(src_hbm.at[s, sb], gidx_v.at[m], msems[m]).wait()

    issue_meta(0, 0)

    zbase = s * ZROWS_A

    @pl.when(s < NS - 1)
    def _():
      pltpu.sync_copy(zrs_hbm, acc_sh.at[pl.ds(zbase, ZROWS_A)])

    @pl.when(s == NS - 1)
    def _():
      pltpu.sync_copy(zrs_hbm.at[pl.ds(0, ZROWS_B)],
                      acc_sh.at[pl.ds(zbase, ZROWS_B)])

    @pl.when(s == 0)
    def _():
      pltpu.sync_copy(rel_hbm, rel_sh)


    def fix_gidx(m, g):
      for k in range(B // LANES):
        sl = pl.ds(k * LANES, LANES)
        gidx_v[m, g, sl] = gidx_v[m, g, sl] * 2 + c

    def fix_sidx(m, g):
      for k in range(B // LANES):
        sl = pl.ds(k * LANES, LANES)
        sidx_v[m, g, sl] = sidx_v[m, g, sl] + ((typ_v[m, g, sl] >> 5) * N_NODES)

    def issue_gather(m, g, b):
      pltpu.async_copy(xh_hbm.at[gidx_v.at[m, g]], xrows_v.at[b], gsems[b])

    def issue_rel_gather(m, g, b):
      for k in range(B // LANES):
        sl = pl.ds(k * LANES, LANES)
        ridx_v[b, sl] = (typ_v[m, g, sl] & (NRH - 1)) * 2 + c
      pltpu.async_copy(rel_sh.at[ridx_v.at[b]], rrows_v.at[b], rsems[b])

    plsc.subcore_barrier()

    def prime_super(m):
      for b in range(2):
        fix_gidx(m, b)
        issue_gather(m, b, b)
        issue_rel_gather(m, b, b)

    def drain_scatter(m, b):
      pltpu.make_async_copy(comp_v.at[b], acc_sh.at[sidx_v.at[m, 0]],
                            ssems[b]).wait()

    def super_body(sb, m, first):
      @pl.loop(0, NSC, step=2)
      def _(g):
        @pl.when(jnp.logical_and(g == 2, sb + 1 < NSUP))
        def _():
          issue_meta(sb + 1, 1 - m)

        for b in range(2):
          gb = g + b
          pltpu.make_async_copy(xh_hbm.at[gidx_v.at[m, gb]], xrows_v.at[b],
                                gsems[b]).wait()
          if first:
            @pl.when(gb >= 2)
            def _():
              drain_scatter(m, b)
          else:
            drain_scatter(m, b)

          pltpu.make_async_copy(rel_sh.at[ridx_v.at[b]], rrows_v.at[b],
                                rsems[b]).wait()

          @pl.loop(0, B // LANES)
          def _(k):
            nvec = nrm_v[m, gb, pl.ds(k * LANES, LANES)]
            for l in range(LANES):
              nv = lax.gather(
                  nvec, jnp.full((LANES, 1), l, jnp.int32),
                  lax.GatherDimensionNumbers(offset_dims=(),
                                             collapsed_slice_dims=(0,),
                                             start_index_map=(0,)),
                  slice_sizes=(1,),
                  mode=lax.GatherScatterMode.PROMISE_IN_BOUNDS)
              i = k * LANES + l
              xs = [xrows_v[b, i, pl.ds(j * LANES, LANES)]
                    for j in range(H // LANES)]
              rs = [rrows_v[b, i, pl.ds(j * LANES, LANES)]
                    for j in range(H // LANES)]
              for j in range(H // LANES):
                comp_v[b, i, pl.ds(j * LANES, LANES)] = (xs[j] * nv) * rs[j]

          @pl.when(gb + 2 < NSC)
          def _():
            fix_gidx(m, gb + 2)
            issue_gather(m, gb + 2, b)
            issue_rel_gather(m, gb + 2, b)

          fix_sidx(m, gb)
          pltpu.async_copy(comp_v.at[b], acc_sh.at[sidx_v.at[m, gb]], ssems[b],
                           add=True)

      @pl.when(sb + 1 < NSUP)
      def _():
        wait_meta(sb + 1, 1 - m)
        prime_super(1 - m)

    wait_meta(0, 0)
    prime_super(0)
    super_body(0, 0, True)

    @pl.loop(1, NSUP - 1, step=2)
    def _(S):
      super_body(S, 1, False)
      super_body(S + 1, 0, False)

    for b in range(2):
      drain_scatter(0, b)

    plsc.subcore_barrier()

    @pl.when(s < NS - 1)
    def _():
      pltpu.sync_copy(acc_sh.at[pl.ds(zbase, ZROWS_A)],
                      out_hbm.at[c, pl.ds(zbase, ZROWS_A)])

    @pl.when(s == NS - 1)
    def _():
      pltpu.sync_copy(acc_sh.at[pl.ds((NS - 1) * ZROWS_A, ZROWS_B)],
                      out_hbm.at[c, pl.ds((NS - 1) * ZROWS_A, ZROWS_B)])

  return sc_kernel(xh, relf, src3, dst3, typ3, nrm3, zrs)


BM = 5000


def _tc_matmul_body(acc_ref, x_ref, w4_ref, wl_ref, o_ref):
  r = jnp.dot(x_ref[...], wl_ref[...], preferred_element_type=jnp.float32)
  for k in range(4):
    r = r + jnp.dot(acc_ref[k], w4_ref[k], preferred_element_type=jnp.float32)

  o_ref[...] = r


def _tc_matmul(acc4, x, w4, w_loop):
  grid = (N_NODES // BM,)
  return pl.pallas_call(
      _tc_matmul_body,
      grid=grid,
      in_specs=[
          pl.BlockSpec((4, BM, H), lambda m: (0, m, 0)),
          pl.BlockSpec((BM, IN_DIM), lambda m: (m, 0)),
          pl.BlockSpec((4, H, OUT_DIM), lambda m: (0, 0, 0)),
          pl.BlockSpec((IN_DIM, OUT_DIM), lambda m: (0, 0)),
      ],
      out_specs=pl.BlockSpec((BM, OUT_DIM), lambda m: (m, 0)),
      out_shape=jax.ShapeDtypeStruct((N_NODES, OUT_DIM), jnp.float32),
  )(acc4, x, w4, w_loop)


@jax.jit
def kernel(x, edge_index, edge_type, edge_norm, w_loop, w_forward, w_backward,
           rel_emb):
  xh = x.reshape(2 * N_NODES, H)
  relf = rel_emb[:NRH].reshape(2 * NRH, H)
  src3 = edge_index[0].reshape(NS, NSUP, NSC, B)
  dst3 = edge_index[1].reshape(NS, NSUP, NSC, B)
  typ3 = edge_type.reshape(NS, NSUP, NSC, B)
  nrm3 = edge_norm.reshape(NS, NSUP, NSC, B)
  zrs = jnp.zeros((ZROWS_A, H), jnp.float32)

  acc = _sc_accumulate(xh, relf, src3, dst3, typ3, nrm3, zrs)
  acc4 = acc.reshape(2, 2, N_NODES, H).reshape(4, N_NODES, H)

  w4 = jnp.stack([w_forward[:H], w_backward[:H],
                  w_forward[H:], w_backward[H:]])

  return _tc_matmul(acc4, x, w4, w_loop)
